# baseline (speedup 1.0000x reference)
import functools

import jax
import jax.numpy as jnp
from jax import lax
from jax.experimental import pallas as pl
from jax.experimental.pallas import tpu as pltpu
from jax.experimental.pallas import tpu_sc as plsc

_NC = 2
_NS = 16
_NW = _NC * _NS
_L = 16
_V = 1000000
_TOTAL = 8 * 16384
_NPW = _TOTAL // _NW
_CHUNK = 128
_NCH = _NPW // _CHUNK
_WPB = _NW // 8
_SWEEP_CH = 16384


def _sweep_body(t_ref, o_ref):
    x = t_ref[...]
    o_ref[...] = jnp.sqrt(jnp.sum(x * x, axis=0, keepdims=True))


_sweep_tc = pl.pallas_call(
    _sweep_body,
    grid=(pl.cdiv(_V, _SWEEP_CH),),
    in_specs=[pl.BlockSpec((_L, _SWEEP_CH), lambda i: (0, i))],
    out_specs=pl.BlockSpec((1, _SWEEP_CH), lambda i: (0, i)),
    out_shape=jax.ShapeDtypeStruct((1, _V), jnp.float32),
)


def _make_gather():
    mesh = plsc.VectorSubcoreMesh(core_axis_name="c", subcore_axis_name="s")

    @functools.partial(
        pl.kernel,
        mesh=mesh,
        compiler_params=pltpu.CompilerParams(needs_layout_passes=False),
        out_type=jax.ShapeDtypeStruct((_NW, _L), jnp.float32),
        scratch_types=[
            pltpu.VMEM((_NCH, _CHUNK), jnp.int32),
            pltpu.VMEM((_NCH, _CHUNK), jnp.float32),
            pltpu.VMEM((_L,), jnp.float32),
            pltpu.SemaphoreType.DMA,
            pltpu.SemaphoreType.DMA,
        ],
    )
    def k(norms_hbm, idx_hbm, out_hbm, idx_v, nrm_v, acc_v, sem_i, sem_g):
        wid = lax.axis_index("s") * _NC + lax.axis_index("c")
        brow = wid // _WPB
        c0 = (wid % _WPB) * _NPW

        idx_copies = [
            pltpu.make_async_copy(
                idx_hbm.at[brow, pl.ds(c0 + j * _CHUNK, _CHUNK)],
                idx_v.at[j], sem_i)
            for j in range(_NCH)
        ]
        for cp in idx_copies:
            cp.start()
        for cp in idx_copies:
            cp.wait()

        gathers = [
            pltpu.make_async_copy(
                norms_hbm.at[idx_v.at[j]], nrm_v.at[j], sem_g)
            for j in range(_NCH)
        ]
        for cp in gathers:
            cp.start()
        for cp in gathers:
            cp.wait()

        def chunk_sum(j, acc):
            for g in range(_CHUNK // _L):
                acc = acc + nrm_v[j, pl.ds(g * _L, _L)]
            return acc

        acc = lax.fori_loop(0, _NCH, chunk_sum,
                            jnp.zeros((_L,), jnp.float32))
        acc_v[...] = acc
        pltpu.sync_copy(acc_v, out_hbm.at[wid])

    return k


_gather_kernel = _make_gather()


def kernel(table, indices):
    norms = _sweep_tc(table.T).reshape(_V)
    partials = _gather_kernel(norms, indices.astype(jnp.int32))
    return jnp.sum(partials) * (1.0 / _TOTAL)
